# Initial kernel scaffold; baseline (speedup 1.0000x reference)
import jax
import jax.numpy as jnp
from jax.experimental import pallas as pl


def kernel(x, edge_index, batch, params):
    raise NotImplementedError("write your pallas kernel here")


# speedup vs baseline: 1.8811x; 1.8811x over previous
import functools

import jax
import jax.numpy as jnp
from jax import lax
from jax.experimental import pallas as pl
from jax.experimental.pallas import tpu as pltpu
from jax.experimental.pallas import tpu_sc as plsc

N_NODES = 10000
N_GRAPH = 64
EB = 128
NSUB = 16
ROWS = 400
GRID = N_NODES // ROWS


def _sc_segsum_call(table, src, dst, zeros, *, f, n_out_pad, rpw, nb):
    t_iters = (nb + NSUB - 1) // NSUB

    def body(table_hbm, src_hbm, dst_hbm, zero_hbm, out_hbm,
             sidx, didx, rows, acc, sem):
        w = lax.axis_index("s")
        pltpu.sync_copy(zero_hbm, acc.at[pl.ds(w * rpw, rpw)])
        plsc.subcore_barrier()

        def step(t, carry):
            b = t * NSUB + w

            @pl.when(b < nb)
            def _():
                off = pl.multiple_of(b * EB, EB)
                pltpu.sync_copy(src_hbm.at[pl.ds(off, EB)], sidx)
                cp = pltpu.async_copy(table_hbm.at[sidx], rows, sem)
                pltpu.sync_copy(dst_hbm.at[pl.ds(off, EB)], didx)
                cp.wait()
                pltpu.sync_copy(rows, acc.at[didx], add=True)

            return carry

        lax.fori_loop(0, t_iters, step, 0)
        plsc.subcore_barrier()
        pltpu.sync_copy(acc.at[pl.ds(w * rpw, rpw)],
                        out_hbm.at[pl.ds(w * rpw, rpw)])

    fn = pl.kernel(
        body,
        out_type=jax.ShapeDtypeStruct((n_out_pad, f), jnp.float32),
        mesh=plsc.VectorSubcoreMesh(core_axis_name="c", subcore_axis_name="s",
                                    num_cores=1),
        scratch_types=[
            pltpu.VMEM((EB,), jnp.int32),
            pltpu.VMEM((EB,), jnp.int32),
            pltpu.VMEM((EB, f), jnp.float32),
            pltpu.VMEM_SHARED((n_out_pad, f), jnp.float32),
            pltpu.SemaphoreType.DMA,
        ],
    )
    return fn(table, src, dst, zeros)


def _stats_body(u_ref, o_ref):
    @pl.when(pl.program_id(0) == 0)
    def _():
        o_ref[...] = jnp.zeros_like(o_ref)

    x = u_ref[...]
    s1 = jnp.sum(x, axis=0, keepdims=True)
    s2 = jnp.sum(x * x, axis=0, keepdims=True)
    o_ref[...] += jnp.concatenate(
        [s1, s2, jnp.zeros((6, x.shape[1]), jnp.float32)], axis=0)


def _tc_stats(u):
    n, c = u.shape
    return pl.pallas_call(
        _stats_body,
        grid=(GRID,),
        in_specs=[pl.BlockSpec((ROWS, c), lambda i: (i, 0))],
        out_specs=pl.BlockSpec((8, c), lambda i: (0, 0)),
        out_shape=jax.ShapeDtypeStruct((8, c), jnp.float32),
    )(u)


def _fused_body(*refs, widths, relu, ones_tail, out_pad, co):
    n_in = len(widths)
    in_refs = refs[:n_in]
    st_ref, w_ref, b_ref, o_ref, m_ref = refs[n_in:]

    @pl.when(pl.program_id(0) == 0)
    def _():
        m_ref[...] = jnp.zeros_like(m_ref)

    z = jnp.concatenate(
        [r[...][:, :w] for r, w in zip(in_refs, widths)], axis=1)
    z = z * st_ref[0:1, :] + st_ref[1:2, :]
    y = jnp.dot(z, w_ref[...], preferred_element_type=jnp.float32)
    y = y + b_ref[0:1, :]
    if relu:
        y = jnp.maximum(y, 0.0)
    pad = out_pad - co
    if pad:
        fill = jnp.ones if ones_tail else jnp.zeros
        y = jnp.concatenate([y, fill((y.shape[0], pad), jnp.float32)], axis=1)
    o_ref[...] = y
    s1 = jnp.sum(y, axis=0, keepdims=True)
    s2 = jnp.sum(y * y, axis=0, keepdims=True)
    m_ref[...] += jnp.concatenate(
        [s1, s2, jnp.zeros((6, out_pad), jnp.float32)], axis=0)


def _tc_fused(inputs, widths, st, w, b, *, relu=True, ones_tail=False,
              out_pad=None):
    cz, co = w.shape
    if out_pad is None:
        out_pad = co
    body = functools.partial(_fused_body, widths=tuple(widths), relu=relu,
                             ones_tail=ones_tail, out_pad=out_pad, co=co)
    in_specs = [pl.BlockSpec((ROWS, u.shape[1]), lambda i: (i, 0))
                for u in inputs]
    in_specs += [
        pl.BlockSpec((8, cz), lambda i: (0, 0)),
        pl.BlockSpec((cz, co), lambda i: (0, 0)),
        pl.BlockSpec((8, co), lambda i: (0, 0)),
    ]
    return pl.pallas_call(
        body,
        grid=(GRID,),
        in_specs=in_specs,
        out_specs=[pl.BlockSpec((ROWS, out_pad), lambda i: (i, 0)),
                   pl.BlockSpec((8, out_pad), lambda i: (0, 0))],
        out_shape=[jax.ShapeDtypeStruct((N_NODES, out_pad), jnp.float32),
                   jax.ShapeDtypeStruct((8, out_pad), jnp.float32)],
    )(*inputs, st, w, b)


def _head_body(p_ref, bn_ref, w1_ref, b1_ref, w2_ref, b2_ref, o_ref):
    s = p_ref[...]
    cnt = jnp.maximum(s[:, 600:601], 1.0)
    h = s[:, :600] / cnt
    mu = jnp.mean(h, axis=0, keepdims=True)
    var = jnp.mean(h * h, axis=0, keepdims=True) - mu * mu
    h = bn_ref[0:1, :] * (h - mu) * lax.rsqrt(var + 1e-5) + bn_ref[1:2, :]
    h = jnp.maximum(jnp.dot(h, w1_ref[...],
                            preferred_element_type=jnp.float32)
                    + b1_ref[0:1, :], 0.0)
    h = jnp.maximum(jnp.dot(h, w2_ref[...],
                            preferred_element_type=jnp.float32)
                    + b2_ref[0:1, :], 0.0)
    h = jnp.dot(h, w2_ref[...], preferred_element_type=jnp.float32) \
        + b2_ref[0:1, :]
    o_ref[...] = h


def _tc_head(psum, bn, w1, b1, w2, b2):
    return pl.pallas_call(
        _head_body,
        out_shape=jax.ShapeDtypeStruct((N_GRAPH, 600), jnp.float32),
    )(psum, bn, w1, b1, w2, b2)


def _row8(v, c=None):
    c = v.shape[0] if c is None else c
    out = jnp.zeros((8, c), jnp.float32)
    return out.at[0, :v.shape[0]].set(v)


def _st_from_moments(moms, gammas, betas):
    s1 = jnp.concatenate([m[0] for m in moms])
    s2 = jnp.concatenate([m[1] for m in moms])
    mu = s1 / N_NODES
    var = s2 / N_NODES - mu * mu
    g = jnp.concatenate(gammas)
    b = jnp.concatenate(betas)
    s = g * lax.rsqrt(var + 1e-5)
    t = b - mu * s
    cz = s.shape[0]
    st = jnp.zeros((8, cz), jnp.float32)
    return st.at[0].set(s).at[1].set(t)


def _segsum_rounds(chunks, src, dst, zeros, nrounds, nb):
    out = []
    for a in chunks:
        for _ in range(nrounds):
            a = _sc_segsum_call(a, src, dst, zeros[:, :a.shape[1]],
                                f=a.shape[1], n_out_pad=10240,
                                rpw=640, nb=nb)
        out.append(a[:N_NODES])
    return out


def kernel(x, edge_index, batch, params):
    src = edge_index[0].astype(jnp.int32)
    dst = edge_index[1].astype(jnp.int32)
    nb_e = src.shape[0] // EB

    zeros640 = jnp.zeros((640, 128), jnp.float32)

    w_e, b_e = params['emb']
    xpad = jnp.pad(x, ((0, 0), (0, 5)))
    wpad = jnp.pad(w_e, ((0, 5), (0, 0)))
    h, m_h = _tc_fused([xpad], [16], _row8(jnp.ones(16)).at[1].set(0.0),
                       wpad, _row8(b_e, 64), relu=False, out_pad=128)

    for li, p in enumerate(params['convs']):
        cin = 64 if li == 0 else 300
        nhops = li + 1
        if li == 0:
            chunks = [h]
            widths_a = [64]
        else:
            chunks = [h[:, :128], h[:, 128:256], h[:, 256:384]]
            widths_a = [128, 128, 44]
        aggs = _segsum_rounds(chunks, src, dst, zeros640, nhops, nb_e)
        m_a = [_tc_stats(a) for a in aggs]
        g1, bb1 = p['bn1_g'], p['bn1_b']
        moms = [m[:, :wd] for m, wd in zip(m_a, widths_a)] + [m_h[:, :cin]]
        st1 = _st_from_moments(moms, [g1[:cin], g1[cin:]], [bb1[:cin],
                                                            bb1[cin:]])
        w1, bv1 = p['lin1']
        u, m_u = _tc_fused(aggs + [h], widths_a + [cin], st1, w1,
                           _row8(bv1, cin))
        st2 = _st_from_moments([m_u[:, :cin]], [p['bn2_g']], [p['bn2_b']])
        w2, bv2 = p['lin2']
        cout = w2.shape[1]
        last = li == len(params['convs']) - 1
        h, m_h = _tc_fused([u], [cin], st2, w2, _row8(bv2, cout),
                           out_pad=640 if last else 384, ones_tail=last)

    e_pool = ((N_NODES + EB - 1) // EB) * EB
    src_p = jnp.concatenate(
        [jnp.arange(N_NODES, dtype=jnp.int32),
         jnp.zeros((e_pool - N_NODES,), jnp.int32)])
    dst_p = jnp.concatenate(
        [batch.astype(jnp.int32),
         jnp.full((e_pool - N_NODES,), N_GRAPH, jnp.int32)])
    zeros8 = jnp.zeros((8, 128), jnp.float32)
    pooled = [
        _sc_segsum_call(h[:, c * 128:(c + 1) * 128], src_p, dst_p, zeros8,
                        f=128, n_out_pad=128, rpw=8, nb=e_pool // EB)
        for c in range(5)
    ]
    psum = jnp.concatenate(pooled, axis=1)[:N_GRAPH]

    bn = jnp.zeros((8, 600), jnp.float32)
    bn = bn.at[0].set(params['bn_g']).at[1].set(params['bn_b'])
    w1, b1 = params['lin1']
    w2, b2 = params['lin2']
    return _tc_head(psum, bn, w1, _row8(b1, 600), w2, _row8(b2, 600))


# speedup vs baseline: 2.6302x; 1.3983x over previous
import functools

import jax
import jax.numpy as jnp
from jax import lax
from jax.experimental import pallas as pl
from jax.experimental.pallas import tpu as pltpu
from jax.experimental.pallas import tpu_sc as plsc

N_NODES = 10000
N_GRAPH = 64
EB = 128
NSUB = 16
ROWS = 400
GRID = N_NODES // ROWS


def _sc_segsum_call(table, src, dst, zeros, *, f, n_out_pad, rpw, nb):
    t_iters = (nb + NSUB - 1) // NSUB
    k_iters = (t_iters + 1) // 2

    def body(table_hbm, src_hbm, dst_hbm, zero_hbm, out_hbm,
             sidx0, sidx1, didx0, didx1, rows0, rows1, acc, sem0, sem1):
        w = lax.axis_index("s")
        pltpu.sync_copy(zero_hbm, acc.at[pl.ds(w * rpw, rpw)])
        plsc.subcore_barrier()

        def fire(b, sidx, rows, sem):
            off = pl.multiple_of(b * EB, EB)
            pltpu.sync_copy(src_hbm.at[pl.ds(off, EB)], sidx)
            pltpu.async_copy(table_hbm.at[sidx], rows, sem)

        def drain(b, sidx, didx, rows, sem):
            off = pl.multiple_of(b * EB, EB)
            pltpu.sync_copy(dst_hbm.at[pl.ds(off, EB)], didx)
            pltpu.make_async_copy(table_hbm.at[sidx], rows, sem).wait()
            pltpu.sync_copy(rows, acc.at[didx], add=True)

        fire(w, sidx0, rows0, sem0)

        def step(k, carry):
            b0 = (2 * k) * NSUB + w
            b1 = (2 * k + 1) * NSUB + w
            b2 = (2 * k + 2) * NSUB + w

            @pl.when(b1 < nb)
            def _():
                fire(b1, sidx1, rows1, sem1)

            @pl.when(b0 < nb)
            def _():
                drain(b0, sidx0, didx0, rows0, sem0)

            @pl.when(b2 < nb)
            def _():
                fire(b2, sidx0, rows0, sem0)

            @pl.when(b1 < nb)
            def _():
                drain(b1, sidx1, didx1, rows1, sem1)

            return carry

        lax.fori_loop(0, k_iters, step, 0)
        plsc.subcore_barrier()
        pltpu.sync_copy(acc.at[pl.ds(w * rpw, rpw)],
                        out_hbm.at[pl.ds(w * rpw, rpw)])

    fn = pl.kernel(
        body,
        out_type=jax.ShapeDtypeStruct((n_out_pad, f), jnp.float32),
        mesh=plsc.VectorSubcoreMesh(core_axis_name="c", subcore_axis_name="s",
                                    num_cores=1),
        scratch_types=[
            pltpu.VMEM((EB,), jnp.int32),
            pltpu.VMEM((EB,), jnp.int32),
            pltpu.VMEM((EB,), jnp.int32),
            pltpu.VMEM((EB,), jnp.int32),
            pltpu.VMEM((EB, f), jnp.float32),
            pltpu.VMEM((EB, f), jnp.float32),
            pltpu.VMEM_SHARED((n_out_pad, f), jnp.float32),
            pltpu.SemaphoreType.DMA,
            pltpu.SemaphoreType.DMA,
        ],
    )
    return fn(table, src, dst, zeros)


def _stats_body(u_ref, o_ref):
    @pl.when(pl.program_id(0) == 0)
    def _():
        o_ref[...] = jnp.zeros_like(o_ref)

    x = u_ref[...]
    s1 = jnp.sum(x, axis=0, keepdims=True)
    s2 = jnp.sum(x * x, axis=0, keepdims=True)
    o_ref[...] += jnp.concatenate(
        [s1, s2, jnp.zeros((6, x.shape[1]), jnp.float32)], axis=0)


def _tc_stats(u):
    n, c = u.shape
    return pl.pallas_call(
        _stats_body,
        grid=(GRID,),
        in_specs=[pl.BlockSpec((ROWS, c), lambda i: (i, 0))],
        out_specs=pl.BlockSpec((8, c), lambda i: (0, 0)),
        out_shape=jax.ShapeDtypeStruct((8, c), jnp.float32),
    )(u)


def _fused_body(*refs, widths, relu, ones_tail, out_pad, co):
    n_in = len(widths)
    in_refs = refs[:n_in]
    st_ref, w_ref, b_ref, o_ref, m_ref = refs[n_in:]

    @pl.when(pl.program_id(0) == 0)
    def _():
        m_ref[...] = jnp.zeros_like(m_ref)

    z = jnp.concatenate(
        [r[...][:, :w] for r, w in zip(in_refs, widths)], axis=1)
    z = z * st_ref[0:1, :] + st_ref[1:2, :]
    y = jnp.dot(z, w_ref[...], preferred_element_type=jnp.float32)
    y = y + b_ref[0:1, :]
    if relu:
        y = jnp.maximum(y, 0.0)
    pad = out_pad - co
    if pad:
        fill = jnp.ones if ones_tail else jnp.zeros
        y = jnp.concatenate([y, fill((y.shape[0], pad), jnp.float32)], axis=1)
    o_ref[...] = y
    s1 = jnp.sum(y, axis=0, keepdims=True)
    s2 = jnp.sum(y * y, axis=0, keepdims=True)
    m_ref[...] += jnp.concatenate(
        [s1, s2, jnp.zeros((6, out_pad), jnp.float32)], axis=0)


def _tc_fused(inputs, widths, st, w, b, *, relu=True, ones_tail=False,
              out_pad=None):
    cz, co = w.shape
    if out_pad is None:
        out_pad = co
    body = functools.partial(_fused_body, widths=tuple(widths), relu=relu,
                             ones_tail=ones_tail, out_pad=out_pad, co=co)
    in_specs = [pl.BlockSpec((ROWS, u.shape[1]), lambda i: (i, 0))
                for u in inputs]
    in_specs += [
        pl.BlockSpec((8, cz), lambda i: (0, 0)),
        pl.BlockSpec((cz, co), lambda i: (0, 0)),
        pl.BlockSpec((8, co), lambda i: (0, 0)),
    ]
    return pl.pallas_call(
        body,
        grid=(GRID,),
        in_specs=in_specs,
        out_specs=[pl.BlockSpec((ROWS, out_pad), lambda i: (i, 0)),
                   pl.BlockSpec((8, out_pad), lambda i: (0, 0))],
        out_shape=[jax.ShapeDtypeStruct((N_NODES, out_pad), jnp.float32),
                   jax.ShapeDtypeStruct((8, out_pad), jnp.float32)],
    )(*inputs, st, w, b)


def _head_body(p_ref, bn_ref, w1_ref, b1_ref, w2_ref, b2_ref, o_ref):
    s = p_ref[...]
    cnt = jnp.maximum(s[:, 600:601], 1.0)
    h = s[:, :600] / cnt
    mu = jnp.mean(h, axis=0, keepdims=True)
    var = jnp.mean(h * h, axis=0, keepdims=True) - mu * mu
    h = bn_ref[0:1, :] * (h - mu) * lax.rsqrt(var + 1e-5) + bn_ref[1:2, :]
    h = jnp.maximum(jnp.dot(h, w1_ref[...],
                            preferred_element_type=jnp.float32)
                    + b1_ref[0:1, :], 0.0)
    h = jnp.maximum(jnp.dot(h, w2_ref[...],
                            preferred_element_type=jnp.float32)
                    + b2_ref[0:1, :], 0.0)
    h = jnp.dot(h, w2_ref[...], preferred_element_type=jnp.float32) \
        + b2_ref[0:1, :]
    o_ref[...] = h


def _tc_head(psum, bn, w1, b1, w2, b2):
    return pl.pallas_call(
        _head_body,
        out_shape=jax.ShapeDtypeStruct((N_GRAPH, 600), jnp.float32),
    )(psum, bn, w1, b1, w2, b2)


def _row8(v, c=None):
    c = v.shape[0] if c is None else c
    out = jnp.zeros((8, c), jnp.float32)
    return out.at[0, :v.shape[0]].set(v)


def _st_from_moments(moms, gammas, betas):
    s1 = jnp.concatenate([m[0] for m in moms])
    s2 = jnp.concatenate([m[1] for m in moms])
    mu = s1 / N_NODES
    var = s2 / N_NODES - mu * mu
    g = jnp.concatenate(gammas)
    b = jnp.concatenate(betas)
    s = g * lax.rsqrt(var + 1e-5)
    t = b - mu * s
    cz = s.shape[0]
    st = jnp.zeros((8, cz), jnp.float32)
    return st.at[0].set(s).at[1].set(t)


def _segsum_rounds(chunks, src, dst, zeros, nrounds, nb):
    out = []
    for a in chunks:
        for _ in range(nrounds):
            a = _sc_segsum_call(a, src, dst, zeros[:, :a.shape[1]],
                                f=a.shape[1], n_out_pad=10240,
                                rpw=640, nb=nb)
        out.append(a[:N_NODES])
    return out


def kernel(x, edge_index, batch, params):
    src = edge_index[0].astype(jnp.int32)
    dst = edge_index[1].astype(jnp.int32)
    nb_e = src.shape[0] // EB

    zeros640 = jnp.zeros((640, 128), jnp.float32)

    w_e, b_e = params['emb']
    xpad = jnp.pad(x, ((0, 0), (0, 5)))
    wpad = jnp.pad(w_e, ((0, 5), (0, 0)))
    h, m_h = _tc_fused([xpad], [16], _row8(jnp.ones(16)).at[1].set(0.0),
                       wpad, _row8(b_e, 64), relu=False, out_pad=128)

    for li, p in enumerate(params['convs']):
        cin = 64 if li == 0 else 300
        nhops = li + 1
        if li == 0:
            chunks = [h]
            widths_a = [64]
        else:
            chunks = [h[:, :128], h[:, 128:256], h[:, 256:384]]
            widths_a = [128, 128, 44]
        aggs = _segsum_rounds(chunks, src, dst, zeros640, nhops, nb_e)
        m_a = [_tc_stats(a) for a in aggs]
        g1, bb1 = p['bn1_g'], p['bn1_b']
        moms = [m[:, :wd] for m, wd in zip(m_a, widths_a)] + [m_h[:, :cin]]
        st1 = _st_from_moments(moms, [g1[:cin], g1[cin:]], [bb1[:cin],
                                                            bb1[cin:]])
        w1, bv1 = p['lin1']
        u, m_u = _tc_fused(aggs + [h], widths_a + [cin], st1, w1,
                           _row8(bv1, cin))
        st2 = _st_from_moments([m_u[:, :cin]], [p['bn2_g']], [p['bn2_b']])
        w2, bv2 = p['lin2']
        cout = w2.shape[1]
        last = li == len(params['convs']) - 1
        h, m_h = _tc_fused([u], [cin], st2, w2, _row8(bv2, cout),
                           out_pad=640 if last else 384, ones_tail=last)

    e_pool = ((N_NODES + EB - 1) // EB) * EB
    src_p = jnp.concatenate(
        [jnp.arange(N_NODES, dtype=jnp.int32),
         jnp.zeros((e_pool - N_NODES,), jnp.int32)])
    dst_p = jnp.concatenate(
        [batch.astype(jnp.int32),
         jnp.full((e_pool - N_NODES,), N_GRAPH, jnp.int32)])
    zeros8 = jnp.zeros((8, 128), jnp.float32)
    pooled = [
        _sc_segsum_call(h[:, c * 128:(c + 1) * 128], src_p, dst_p, zeros8,
                        f=128, n_out_pad=128, rpw=8, nb=e_pool // EB)
        for c in range(5)
    ]
    psum = jnp.concatenate(pooled, axis=1)[:N_GRAPH]

    bn = jnp.zeros((8, 600), jnp.float32)
    bn = bn.at[0].set(params['bn_g']).at[1].set(params['bn_b'])
    w1, b1 = params['lin1']
    w2, b2 = params['lin2']
    return _tc_head(psum, bn, w1, _row8(b1, 600), w2, _row8(b2, 600))
